# Initial kernel scaffold; baseline (speedup 1.0000x reference)
import jax
import jax.numpy as jnp
from jax.experimental import pallas as pl


def kernel(spacing_bins, weights, nears, fars):
    raise NotImplementedError("write your pallas kernel here")


# speedup vs baseline: 2.6127x; 2.6127x over previous
import functools

import jax
import jax.numpy as jnp
from jax import lax
from jax.experimental import pallas as pl
from jax.experimental.pallas import tpu as pltpu
from jax.experimental.pallas import tpu_sc as plsc

R = 65536
S = 64
NB = S + 1
OUT = 2 * S + 1
L = 16
NW = 32
RAYS_PER_W = R // NW
GROUPS = RAYS_PER_W // L
HIST_PAD = 0.01
EPS = 1e-5


def _body(bins_hbm, w_hbm, near_hbm, far_hbm, u_hbm,
          out_hbm,
          bins_v, w_v, near_v, far_v, u_v, cdf_v, newb_v, out_v):
    wid = lax.axis_index("s") * 2 + lax.axis_index("c")
    iota = lax.iota(jnp.int32, L)
    pltpu.sync_copy(u_hbm, u_v)

    def group(g, _):
        base = wid * RAYS_PER_W + g * L
        pltpu.sync_copy(bins_hbm.at[pl.ds(base, L)], bins_v)
        pltpu.sync_copy(w_hbm.at[pl.ds(base, L)], w_v)
        pltpu.sync_copy(near_hbm.at[pl.ds(base, L)], near_v)
        pltpu.sync_copy(far_hbm.at[pl.ds(base, L)], far_v)

        def col(ref, c):
            return plsc.load_gather(ref, [iota, jnp.full((L,), c, jnp.int32)])

        def suma(k, acc):
            return acc + (col(w_v, k) + HIST_PAD)
        wsum = lax.fori_loop(0, S, suma, jnp.zeros((L,), jnp.float32))
        pad = jnp.maximum(jnp.float32(EPS) - wsum, 0.0)
        wadd = pad * jnp.float32(1.0 / S)
        inv = 1.0 / (wsum + pad)

        zeros = jnp.zeros((L,), jnp.float32)
        plsc.store_scatter(cdf_v, [iota, jnp.full((L,), 0, jnp.int32)], zeros)
        plsc.store_scatter(cdf_v, [iota, jnp.full((L,), NB, jnp.int32)],
                           jnp.full((L,), 2.0, jnp.float32))

        def cuml(k, csum):
            csum = csum + (col(w_v, k) + HIST_PAD + wadd) * inv
            plsc.store_scatter(cdf_v, [iota, jnp.full((L,), k + 1, jnp.int32)],
                               jnp.minimum(csum, 1.0))
            return csum
        lax.fori_loop(0, S, cuml, zeros)

        def sample(j, kvec):
            uj = plsc.load_gather(u_v, [jnp.full((L,), j, jnp.int32)])

            def adv_cond(kv):
                c1 = plsc.load_gather(cdf_v, [iota, kv + 1])
                return jnp.any((kv < S) & (c1 <= uj))

            def adv_body(kv):
                c1 = plsc.load_gather(cdf_v, [iota, kv + 1])
                return kv + ((kv < S) & (c1 <= uj)).astype(jnp.int32)

            kvec = lax.while_loop(adv_cond, adv_body, kvec)
            c0 = plsc.load_gather(cdf_v, [iota, kvec])
            c1 = plsc.load_gather(cdf_v, [iota, kvec + 1])
            b0 = plsc.load_gather(bins_v, [iota, kvec])
            b1 = plsc.load_gather(bins_v, [iota, jnp.minimum(kvec + 1, S)])
            t = jnp.clip((uj - c0) / (c1 - c0), 0.0, 1.0)
            plsc.store_scatter(newb_v, [iota, jnp.full((L,), j, jnp.int32)],
                               b0 + t * (b1 - b0))
            return kvec
        lax.fori_loop(0, NB, sample, jnp.zeros((L,), jnp.int32))

        near = near_v[...]
        span = far_v[...] - near
        big = jnp.float32(jnp.inf)

        def merge(p, iv):
            i_vec, j_vec = iv
            a = plsc.load_gather(bins_v, [iota, i_vec])
            b = plsc.load_gather(newb_v, [iota, j_vec])
            a = jnp.where(i_vec >= S, big, a)
            b = jnp.where(j_vec >= S, big, b)
            take = a <= b
            m = jnp.minimum(a, b)
            plsc.store_scatter(out_v, [iota, jnp.full((L,), p, jnp.int32)],
                               near + m * span)
            i_vec = i_vec + take.astype(jnp.int32)
            j_vec = j_vec + (~take).astype(jnp.int32)
            return i_vec, j_vec
        z = jnp.zeros((L,), jnp.int32)
        lax.fori_loop(0, 2 * S, merge, (z, z))

        ends = jnp.maximum(col(bins_v, S), col(newb_v, S))
        plsc.store_scatter(out_v, [iota, jnp.full((L,), 2 * S, jnp.int32)],
                           near + ends * span)

        pltpu.sync_copy(out_v, out_hbm.at[pl.ds(base, L)])
        return 0

    lax.fori_loop(0, GROUPS, group, 0)


@jax.jit
def kernel(spacing_bins, weights, nears, fars):
    u = (jnp.linspace(0.0, 1.0 - 1.0 / NB, NB, dtype=jnp.float32)
         + 1.0 / (2 * NB))
    u_pad = jnp.zeros((80,), jnp.float32).at[:NB].set(u)
    mesh = plsc.VectorSubcoreMesh(core_axis_name="c", subcore_axis_name="s")
    fn = pl.kernel(
        _body,
        out_type=jax.ShapeDtypeStruct((R, OUT), jnp.float32),
        mesh=mesh,
        compiler_params=pltpu.CompilerParams(needs_layout_passes=False),
        scratch_types=[
            pltpu.VMEM((L, NB), jnp.float32),
            pltpu.VMEM((L, S), jnp.float32),
            pltpu.VMEM((L,), jnp.float32),
            pltpu.VMEM((L,), jnp.float32),
            pltpu.VMEM((80,), jnp.float32),
            pltpu.VMEM((L, NB + 1), jnp.float32),
            pltpu.VMEM((L, NB), jnp.float32),
            pltpu.VMEM((L, OUT), jnp.float32),
        ],
    )
    return fn(spacing_bins, weights, nears.reshape(R), fars.reshape(R), u_pad)


# speedup vs baseline: 4.4598x; 1.7070x over previous
import jax
import jax.numpy as jnp
from jax import lax
from jax.experimental import pallas as pl
from jax.experimental.pallas import tpu as pltpu
from jax.experimental.pallas import tpu_sc as plsc

R = 65536
S = 64
NB = S + 1
OUT = 2 * S + 1
L = 16
NW = 32
SLAB = 128
GPS = SLAB // L
SLABS = R // NW // SLAB
HIST_PAD = 0.01
EPS = 1e-5


def _body(bins_hbm, w_hbm, near_hbm, far_hbm, u_hbm,
          out_hbm,
          bins_v, w_v, near_v, far_v, u_v, cdf_v, newb_v, out_v):
    wid = lax.axis_index("s") * 2 + lax.axis_index("c")
    iota = lax.iota(jnp.int32, L)
    pltpu.sync_copy(u_hbm, u_v)

    def slab(sl, _):
        base = wid * (SLABS * SLAB) + sl * SLAB
        pltpu.sync_copy(bins_hbm.at[pl.ds(base, SLAB)], bins_v)
        pltpu.sync_copy(w_hbm.at[pl.ds(base, SLAB)], w_v)
        pltpu.sync_copy(near_hbm.at[pl.ds(base, SLAB)], near_v)
        pltpu.sync_copy(far_hbm.at[pl.ds(base, SLAB)], far_v)

        def group(g, _):
            rowv = iota + g * L

            def col(ref, c):
                return plsc.load_gather(ref, [rowv, jnp.full((L,), c, jnp.int32)])

            def suma(k, acc):
                return acc + col(w_v, k)
            wsum = lax.fori_loop(0, S, suma, jnp.zeros((L,), jnp.float32),
                                 unroll=8) + jnp.float32(S * HIST_PAD)
            pad = jnp.maximum(jnp.float32(EPS) - wsum, 0.0)
            wadd = pad * jnp.float32(1.0 / S)
            inv = 1.0 / (wsum + pad)

            zeros = jnp.zeros((L,), jnp.float32)
            plsc.store_scatter(cdf_v, [iota, jnp.full((L,), 0, jnp.int32)], zeros)
            plsc.store_scatter(cdf_v, [iota, jnp.full((L,), NB, jnp.int32)],
                               jnp.full((L,), 2.0, jnp.float32))

            def cuml(k, csum):
                csum = csum + (col(w_v, k) + (HIST_PAD + wadd)) * inv
                plsc.store_scatter(cdf_v, [iota, jnp.full((L,), k + 1, jnp.int32)],
                                   jnp.minimum(csum, 1.0))
                return csum
            lax.fori_loop(0, S, cuml, zeros, unroll=8)

            def step(_, st):
                kvec, jvec, c0 = st
                uj = plsc.load_gather(u_v, [jvec])
                c1 = plsc.load_gather(cdf_v, [iota, kvec + 1])
                live = jvec < NB
                adv = (c1 <= uj) & (kvec < S) & live
                emit = (~adv) & live
                b0 = plsc.load_gather(bins_v, [rowv, kvec])
                b1 = plsc.load_gather(bins_v, [rowv, jnp.minimum(kvec + 1, S)])
                t = jnp.clip((uj - c0) / (c1 - c0), 0.0, 1.0)
                plsc.store_scatter(newb_v, [iota, jnp.minimum(jvec, NB - 1)],
                                   b0 + t * (b1 - b0), mask=emit)
                kvec = kvec + adv.astype(jnp.int32)
                jvec = jvec + emit.astype(jnp.int32)
                c0 = jnp.where(adv, c1, c0)
                return kvec, jvec, c0
            z = jnp.zeros((L,), jnp.int32)
            lax.fori_loop(0, S + NB + 1, step, (z, z, jnp.zeros((L,), jnp.float32)),
                          unroll=5)

            near = plsc.load_gather(near_v, [rowv])
            span = plsc.load_gather(far_v, [rowv]) - near
            big = jnp.float32(jnp.inf)

            def merge(p, iv):
                i_vec, j_vec = iv
                a = plsc.load_gather(bins_v, [rowv, i_vec])
                b = plsc.load_gather(newb_v, [iota, j_vec])
                a = jnp.where(i_vec >= S, big, a)
                b = jnp.where(j_vec >= S, big, b)
                take = a <= b
                m = jnp.minimum(a, b)
                plsc.store_scatter(out_v, [rowv, jnp.full((L,), p, jnp.int32)],
                                   near + m * span)
                i_vec = i_vec + take.astype(jnp.int32)
                j_vec = j_vec + (~take).astype(jnp.int32)
                return i_vec, j_vec
            lax.fori_loop(0, 2 * S, merge, (z, z), unroll=4)

            ends = jnp.maximum(
                col(bins_v, S),
                plsc.load_gather(newb_v, [iota, jnp.full((L,), S, jnp.int32)]))
            plsc.store_scatter(out_v, [rowv, jnp.full((L,), 2 * S, jnp.int32)],
                               near + ends * span)
            return 0

        lax.fori_loop(0, GPS, group, 0)
        pltpu.sync_copy(out_v, out_hbm.at[pl.ds(base, SLAB)])
        return 0

    lax.fori_loop(0, SLABS, slab, 0)


@jax.jit
def kernel(spacing_bins, weights, nears, fars):
    u = (jnp.linspace(0.0, 1.0 - 1.0 / NB, NB, dtype=jnp.float32)
         + 1.0 / (2 * NB))
    u_pad = jnp.zeros((80,), jnp.float32).at[:NB].set(u)
    mesh = plsc.VectorSubcoreMesh(core_axis_name="c", subcore_axis_name="s")
    fn = pl.kernel(
        _body,
        out_type=jax.ShapeDtypeStruct((R, OUT), jnp.float32),
        mesh=mesh,
        compiler_params=pltpu.CompilerParams(needs_layout_passes=False),
        scratch_types=[
            pltpu.VMEM((SLAB, NB), jnp.float32),
            pltpu.VMEM((SLAB, S), jnp.float32),
            pltpu.VMEM((SLAB,), jnp.float32),
            pltpu.VMEM((SLAB,), jnp.float32),
            pltpu.VMEM((80,), jnp.float32),
            pltpu.VMEM((L, NB + 1), jnp.float32),
            pltpu.VMEM((L, NB), jnp.float32),
            pltpu.VMEM((SLAB, OUT), jnp.float32),
        ],
    )
    return fn(spacing_bins, weights, nears.reshape(R), fars.reshape(R), u_pad)


# speedup vs baseline: 5.4871x; 1.2304x over previous
import jax
import jax.numpy as jnp
from jax import lax
from jax.experimental import pallas as pl
from jax.experimental.pallas import tpu as pltpu
from jax.experimental.pallas import tpu_sc as plsc

R = 65536
S = 64
NB = S + 1
OUT = 2 * S + 1
L = 16
NW = 32
SLAB = 128
PAIRS = SLAB // (2 * L)
SLABS = R // NW // SLAB
HIST_PAD = 0.01


def _body(bins_hbm, w_hbm, near_hbm, far_hbm, u_hbm,
          out_hbm,
          bins_v, w_v, near_v, far_v, u_v, cdf_v, newb_v, out_v):
    wid = lax.axis_index("s") * 2 + lax.axis_index("c")
    iota = lax.iota(jnp.int32, L)
    iota2 = iota + L
    zf = jnp.zeros((L,), jnp.float32)
    zi = jnp.zeros((L,), jnp.int32)
    pltpu.sync_copy(u_hbm, u_v)

    def slab(sl, _):
        base = wid * (SLABS * SLAB) + sl * SLAB
        pltpu.sync_copy(bins_hbm.at[pl.ds(base, SLAB)], bins_v)
        pltpu.sync_copy(w_hbm.at[pl.ds(base, SLAB)], w_v)
        pltpu.sync_copy(near_hbm.at[pl.ds(base, SLAB)], near_v)
        pltpu.sync_copy(far_hbm.at[pl.ds(base, SLAB)], far_v)

        def pair(p, _):
            row0 = iota + p * (2 * L)
            row1 = row0 + L

            def cum(k, cs):
                fk = jnp.full((L,), k, jnp.int32)
                v0 = plsc.load_gather(w_v, [row0, fk])
                v1 = plsc.load_gather(w_v, [row1, fk])
                cs0 = cs[0] + (v0 + HIST_PAD)
                cs1 = cs[1] + (v1 + HIST_PAD)
                plsc.store_scatter(cdf_v, [iota, fk + 1], cs0)
                plsc.store_scatter(cdf_v, [iota2, fk + 1], cs1)
                return cs0, cs1
            wsum0, wsum1 = lax.fori_loop(0, S, cum, (zf, zf), unroll=8)
            f0 = jnp.full((L,), 0, jnp.int32)
            fNB = jnp.full((L,), NB, jnp.int32)
            plsc.store_scatter(cdf_v, [iota, f0], zf)
            plsc.store_scatter(cdf_v, [iota2, f0], zf)
            plsc.store_scatter(cdf_v, [iota, fNB], wsum0 + 1.0)
            plsc.store_scatter(cdf_v, [iota2, fNB], wsum1 + 1.0)

            def step(_, st):
                k0, j0, a0, k1, j1, a1 = st
                uw0 = plsc.load_gather(u_v, [j0]) * wsum0
                uw1 = plsc.load_gather(u_v, [j1]) * wsum1
                c10 = plsc.load_gather(cdf_v, [iota, k0 + 1])
                c11 = plsc.load_gather(cdf_v, [iota2, k1 + 1])
                adv0 = (c10 <= uw0) & (k0 < S) & (j0 < NB)
                adv1 = (c11 <= uw1) & (k1 < S) & (j1 < NB)
                emit0 = (~adv0) & (j0 < NB)
                emit1 = (~adv1) & (j1 < NB)
                b00 = plsc.load_gather(bins_v, [row0, k0])
                b01 = plsc.load_gather(bins_v, [row0, jnp.minimum(k0 + 1, S)])
                b10 = plsc.load_gather(bins_v, [row1, k1])
                b11 = plsc.load_gather(bins_v, [row1, jnp.minimum(k1 + 1, S)])
                t0 = jnp.clip((uw0 - a0) / (c10 - a0), 0.0, 1.0)
                t1 = jnp.clip((uw1 - a1) / (c11 - a1), 0.0, 1.0)
                plsc.store_scatter(newb_v, [iota, jnp.minimum(j0, NB - 1)],
                                   b00 + t0 * (b01 - b00), mask=emit0)
                plsc.store_scatter(newb_v, [iota2, jnp.minimum(j1, NB - 1)],
                                   b10 + t1 * (b11 - b10), mask=emit1)
                return (k0 + adv0.astype(jnp.int32), j0 + emit0.astype(jnp.int32),
                        jnp.where(adv0, c10, a0),
                        k1 + adv1.astype(jnp.int32), j1 + emit1.astype(jnp.int32),
                        jnp.where(adv1, c11, a1))
            lax.fori_loop(0, S + NB + 1, step, (zi, zi, zf, zi, zi, zf),
                          unroll=5)

            near0 = plsc.load_gather(near_v, [row0])
            near1 = plsc.load_gather(near_v, [row1])
            span0 = plsc.load_gather(far_v, [row0]) - near0
            span1 = plsc.load_gather(far_v, [row1]) - near1
            big = jnp.float32(jnp.inf)

            def merge(pp, st):
                i0, j0, i1, j1 = st
                fp = jnp.full((L,), pp, jnp.int32)
                a0 = plsc.load_gather(bins_v, [row0, i0])
                b0 = plsc.load_gather(newb_v, [iota, j0])
                a1 = plsc.load_gather(bins_v, [row1, i1])
                b1 = plsc.load_gather(newb_v, [iota2, j1])
                a0 = jnp.where(i0 >= S, big, a0)
                b0 = jnp.where(j0 >= S, big, b0)
                a1 = jnp.where(i1 >= S, big, a1)
                b1 = jnp.where(j1 >= S, big, b1)
                take0 = a0 <= b0
                take1 = a1 <= b1
                plsc.store_scatter(out_v, [row0, fp],
                                   near0 + jnp.minimum(a0, b0) * span0)
                plsc.store_scatter(out_v, [row1, fp],
                                   near1 + jnp.minimum(a1, b1) * span1)
                return (i0 + take0.astype(jnp.int32), j0 + (~take0).astype(jnp.int32),
                        i1 + take1.astype(jnp.int32), j1 + (~take1).astype(jnp.int32))
            lax.fori_loop(0, 2 * S, merge, (zi, zi, zi, zi), unroll=4)

            fS = jnp.full((L,), S, jnp.int32)
            fO = jnp.full((L,), 2 * S, jnp.int32)
            ends0 = jnp.maximum(plsc.load_gather(bins_v, [row0, fS]),
                                plsc.load_gather(newb_v, [iota, fS]))
            ends1 = jnp.maximum(plsc.load_gather(bins_v, [row1, fS]),
                                plsc.load_gather(newb_v, [iota2, fS]))
            plsc.store_scatter(out_v, [row0, fO], near0 + ends0 * span0)
            plsc.store_scatter(out_v, [row1, fO], near1 + ends1 * span1)
            return 0

        lax.fori_loop(0, PAIRS, pair, 0)
        pltpu.sync_copy(out_v, out_hbm.at[pl.ds(base, SLAB)])
        return 0

    lax.fori_loop(0, SLABS, slab, 0)


@jax.jit
def kernel(spacing_bins, weights, nears, fars):
    u = (jnp.linspace(0.0, 1.0 - 1.0 / NB, NB, dtype=jnp.float32)
         + 1.0 / (2 * NB))
    u_pad = jnp.zeros((80,), jnp.float32).at[:NB].set(u)
    mesh = plsc.VectorSubcoreMesh(core_axis_name="c", subcore_axis_name="s")
    fn = pl.kernel(
        _body,
        out_type=jax.ShapeDtypeStruct((R, OUT), jnp.float32),
        mesh=mesh,
        compiler_params=pltpu.CompilerParams(needs_layout_passes=False),
        scratch_types=[
            pltpu.VMEM((SLAB, NB), jnp.float32),
            pltpu.VMEM((SLAB, S), jnp.float32),
            pltpu.VMEM((SLAB,), jnp.float32),
            pltpu.VMEM((SLAB,), jnp.float32),
            pltpu.VMEM((80,), jnp.float32),
            pltpu.VMEM((2 * L, NB + 1), jnp.float32),
            pltpu.VMEM((2 * L, NB), jnp.float32),
            pltpu.VMEM((SLAB, OUT), jnp.float32),
        ],
    )
    return fn(spacing_bins, weights, nears.reshape(R), fars.reshape(R), u_pad)
